# baseline (speedup 1.0000x reference)
import jax
import jax.numpy as jnp
from jax import lax
from jax.experimental import pallas as pl
from jax.experimental.pallas import tpu as pltpu
from jax.experimental.pallas import tpu_sc as plsc

N = 10000
E = 320000
DIN = 128
DH = 128
DOUT = 64
G = 64

NC = 2
NS = 16
NW = NC * NS
EC = E // NW
CHUNK = 128
NFULL = EC // CHUNK
TAIL = EC - NFULL * CHUNK
NPAD = 10240
RPT = NPAD // NS
BLK = 2000

_mesh = plsc.VectorSubcoreMesh(core_axis_name="c", subcore_axis_name="s")


def _deg_body(dst_hbm, out_hbm, dst_v, deg_v):
    c = lax.axis_index("c")
    s = lax.axis_index("s")
    wid = s * NC + c
    base = wid * EC
    pltpu.sync_copy(dst_hbm.at[pl.ds(base, EC)], dst_v)
    z16 = jnp.zeros((16,), jnp.float32)
    ones16 = jnp.ones((16,), jnp.float32)

    def zero_body(i, carry):
        deg_v[pl.ds(i * 16, 16)] = z16
        return carry

    lax.fori_loop(0, NPAD // 16, zero_body, 0)

    def acc_body(i, carry):
        idx = dst_v[pl.ds(i * 16, 16)]
        plsc.addupdate_scatter(deg_v, [idx], ones16)
        return carry

    lax.fori_loop(0, EC // 16, acc_body, 0)
    pltpu.sync_copy(deg_v, out_hbm.at[wid])


_deg_call = pl.kernel(
    _deg_body,
    out_type=jax.ShapeDtypeStruct((NW, NPAD), jnp.float32),
    mesh=_mesh,
    compiler_params=pltpu.CompilerParams(needs_layout_passes=False),
    scratch_types=[
        pltpu.VMEM((EC,), jnp.int32),
        pltpu.VMEM((NPAD,), jnp.float32),
    ],
)


def _edge_body(src_hbm, dst_hbm, y_hbm, out_hbm, z_sh, sidx0, didx0,
               sidx1, didx1, rows0, rows1, sidx_t, didx_t, rows_t,
               gsem0, gsem1, ssem0, ssem1):
    c = lax.axis_index("c")
    s = lax.axis_index("s")
    wid = s * NC + c
    base = wid * EC

    z16 = jnp.zeros((16,), jnp.float32)

    def zrow(j, carry):
        def zcol(k, c2):
            rows0[j, pl.ds(k * 16, 16)] = z16
            return c2

        lax.fori_loop(0, DH // 16, zcol, 0)
        return carry

    lax.fori_loop(0, CHUNK, zrow, 0)
    row0 = s * RPT
    for r in range(RPT // CHUNK):
        pltpu.sync_copy(rows0, z_sh.at[pl.ds(row0 + r * CHUNK, CHUNK)])

    plsc.subcore_barrier()

    def chunk_body(i, carry):
        off = base + (2 * i) * CHUNK
        pltpu.sync_copy(src_hbm.at[pl.ds(off, CHUNK)], sidx0)
        pltpu.sync_copy(dst_hbm.at[pl.ds(off, CHUNK)], didx0)
        cp0 = pltpu.async_copy(y_hbm.at[sidx0], rows0, gsem0)
        pltpu.sync_copy(src_hbm.at[pl.ds(off + CHUNK, CHUNK)], sidx1)
        pltpu.sync_copy(dst_hbm.at[pl.ds(off + CHUNK, CHUNK)], didx1)
        cp1 = pltpu.async_copy(y_hbm.at[sidx1], rows1, gsem1)
        cp0.wait()
        sc0 = pltpu.async_copy(rows0, z_sh.at[didx0], ssem0, add=True)
        cp1.wait()
        sc1 = pltpu.async_copy(rows1, z_sh.at[didx1], ssem1, add=True)
        sc0.wait()
        sc1.wait()
        return carry

    lax.fori_loop(0, NFULL // 2, chunk_body, 0)

    off = base + NFULL * CHUNK
    pltpu.sync_copy(src_hbm.at[pl.ds(off, TAIL)], sidx_t)
    cp = pltpu.async_copy(y_hbm.at[sidx_t], rows_t, gsem0)
    pltpu.sync_copy(dst_hbm.at[pl.ds(off, TAIL)], didx_t)
    cp.wait()
    pltpu.sync_copy(rows_t, z_sh.at[didx_t], add=True)

    plsc.subcore_barrier()
    pltpu.sync_copy(z_sh.at[pl.ds(row0, RPT)], out_hbm.at[c, pl.ds(row0, RPT)])


_edge_call = pl.kernel(
    _edge_body,
    out_type=jax.ShapeDtypeStruct((NC, NPAD, DH), jnp.float32),
    mesh=_mesh,
    scratch_types=[
        pltpu.VMEM_SHARED((NPAD, DH), jnp.float32),
        pltpu.VMEM((CHUNK,), jnp.int32),
        pltpu.VMEM((CHUNK,), jnp.int32),
        pltpu.VMEM((CHUNK,), jnp.int32),
        pltpu.VMEM((CHUNK,), jnp.int32),
        pltpu.VMEM((CHUNK, DH), jnp.float32),
        pltpu.VMEM((CHUNK, DH), jnp.float32),
        pltpu.VMEM((TAIL,), jnp.int32),
        pltpu.VMEM((TAIL,), jnp.int32),
        pltpu.VMEM((TAIL, DH), jnp.float32),
        pltpu.SemaphoreType.DMA,
        pltpu.SemaphoreType.DMA,
        pltpu.SemaphoreType.DMA,
        pltpu.SemaphoreType.DMA,
    ],
)


def _tca_body(x_ref, w_ref, degt_ref, y_ref):
    deg = jnp.sum(degt_ref[...], axis=1, keepdims=True) + 1.0
    dinv = lax.rsqrt(deg)
    xw = jnp.dot(x_ref[...], w_ref[...], preferred_element_type=jnp.float32)
    y_ref[...] = xw * dinv


def _tca(x, w, degt):
    return pl.pallas_call(
        _tca_body,
        grid=(N // BLK,),
        in_specs=[
            pl.BlockSpec((BLK, DIN), lambda i: (i, 0)),
            pl.BlockSpec((DIN, DH), lambda i: (0, 0)),
            pl.BlockSpec((BLK, NW), lambda i: (i, 0)),
        ],
        out_specs=pl.BlockSpec((BLK, DH), lambda i: (i, 0)),
        out_shape=jax.ShapeDtypeStruct((NPAD, DH), jnp.float32),
    )(x, w, degt)


def _tcb_body(z_ref, y_ref, degt_ref, bconv_ref, batch_ref,
              wlin_ref, blin_ref, out_ref, sums, cnt):
    i = pl.program_id(0)

    @pl.when(i == 0)
    def _():
        sums[...] = jnp.zeros_like(sums)
        cnt[...] = jnp.zeros_like(cnt)

    deg = jnp.sum(degt_ref[...], axis=1, keepdims=True) + 1.0
    dinv = lax.rsqrt(deg)
    z = z_ref[0] + z_ref[1]
    h = dinv * (z + y_ref[...]) + bconv_ref[...]
    h = jnp.maximum(h, 0.0)
    gid = lax.broadcasted_iota(jnp.int32, (BLK, G), 1)
    onehot = (batch_ref[...] == gid).astype(jnp.float32)
    sums[...] += lax.dot_general(onehot, h, (((0,), (0,)), ((), ())),
                                 preferred_element_type=jnp.float32)
    cnt[...] += lax.dot_general(onehot, jnp.ones((BLK, 1), jnp.float32),
                                (((0,), (0,)), ((), ())),
                                preferred_element_type=jnp.float32)

    @pl.when(i == pl.num_programs(0) - 1)
    def _():
        emb = sums[...] / jnp.maximum(cnt[...], 1.0)
        out_ref[...] = jnp.tanh(
            jnp.dot(emb, wlin_ref[...], preferred_element_type=jnp.float32)
            + blin_ref[...])


def _tcb(zp, y, degt, bconv, batch2, wlin, blin):
    return pl.pallas_call(
        _tcb_body,
        grid=(N // BLK,),
        in_specs=[
            pl.BlockSpec((NC, BLK, DH), lambda i: (0, i, 0)),
            pl.BlockSpec((BLK, DH), lambda i: (i, 0)),
            pl.BlockSpec((BLK, NW), lambda i: (i, 0)),
            pl.BlockSpec((1, DH), lambda i: (0, 0)),
            pl.BlockSpec((BLK, 1), lambda i: (i, 0)),
            pl.BlockSpec((DH, DOUT), lambda i: (0, 0)),
            pl.BlockSpec((1, DOUT), lambda i: (0, 0)),
        ],
        out_specs=pl.BlockSpec((G, DOUT), lambda i: (0, 0)),
        out_shape=jax.ShapeDtypeStruct((G, DOUT), jnp.float32),
        scratch_shapes=[
            pltpu.VMEM((G, DH), jnp.float32),
            pltpu.VMEM((G, 1), jnp.float32),
        ],
    )(zp, y, degt, bconv, batch2, wlin, blin)


@jax.jit
def kernel(x, edge_index, batch, W_conv, b_conv, W_lin, b_lin):
    src = edge_index[0].astype(jnp.int32)
    dst = edge_index[1].astype(jnp.int32)

    degp = _deg_call(dst)
    degt = degp.T

    y = _tca(x, W_conv, degt)

    zp = _edge_call(src, dst, y)

    batch2 = batch.astype(jnp.int32).reshape(N, 1)
    bconv = b_conv.reshape(1, DH)
    blin = b_lin.reshape(1, DOUT)
    return _tcb(zp, y, degt, bconv, batch2, W_lin, blin)


# speedup vs baseline: 1.0991x; 1.0991x over previous
import jax
import jax.numpy as jnp
from jax import lax
from jax.experimental import pallas as pl
from jax.experimental.pallas import tpu as pltpu
from jax.experimental.pallas import tpu_sc as plsc

N = 10000
E = 320000
DIN = 128
DH = 128
DOUT = 64
G = 64

NC = 2
NS = 16
NW = NC * NS
EC = E // NW
CHUNK = 128
NFULL = EC // CHUNK
TAIL = EC - NFULL * CHUNK
PCH = NFULL // 2
PH = PCH * CHUNK
NPAD = 10240
RPT = NPAD // NS
BLK = 2000

_mesh = plsc.VectorSubcoreMesh(core_axis_name="c", subcore_axis_name="s")


def _deg_body(dst_hbm, out_hbm, dst_v, deg_v):
    c = lax.axis_index("c")
    s = lax.axis_index("s")
    wid = s * NC + c
    base = wid * EC
    pltpu.sync_copy(dst_hbm.at[pl.ds(base, EC)], dst_v)
    z16 = jnp.zeros((16,), jnp.float32)
    ones16 = jnp.ones((16,), jnp.float32)

    def zero_body(i, carry):
        deg_v[pl.ds(i * 16, 16)] = z16
        return carry

    lax.fori_loop(0, NPAD // 16, zero_body, 0)

    def acc_body(i, carry):
        idx = dst_v[pl.ds(i * 16, 16)]
        plsc.addupdate_scatter(deg_v, [idx], ones16)
        return carry

    lax.fori_loop(0, EC // 16, acc_body, 0)
    pltpu.sync_copy(deg_v, out_hbm.at[wid])


_deg_call = pl.kernel(
    _deg_body,
    out_type=jax.ShapeDtypeStruct((NW, NPAD), jnp.float32),
    mesh=_mesh,
    compiler_params=pltpu.CompilerParams(needs_layout_passes=False),
    scratch_types=[
        pltpu.VMEM((EC,), jnp.int32),
        pltpu.VMEM((NPAD,), jnp.float32),
    ],
)


def _edge_body(src_hbm, dst_hbm, y_hbm, out_hbm, z_sh, sall, dall,
               sidx0, didx0, sidx1, didx1, rows0, rows1,
               sidx_t, didx_t, rows_t, gsem0, gsem1, ssem0, ssem1):
    c = lax.axis_index("c")
    s = lax.axis_index("s")
    wid = s * NC + c
    base = wid * EC

    z16 = jnp.zeros((16,), jnp.float32)

    def zrow(j, carry):
        def zcol(k, c2):
            rows0[j, pl.ds(k * 16, 16)] = z16
            return c2

        lax.fori_loop(0, DH // 16, zcol, 0)
        return carry

    lax.fori_loop(0, CHUNK, zrow, 0)
    row0 = s * RPT
    for r in range(RPT // CHUNK):
        pltpu.sync_copy(rows0, z_sh.at[pl.ds(row0 + r * CHUNK, CHUNK)])

    plsc.subcore_barrier()

    def regcopy(dst_small, src_big, off):
        for k in range(CHUNK // 16):
            dst_small[pl.ds(k * 16, 16)] = src_big[pl.ds(off + k * 16, 16)]

    for p in range(2):
        pltpu.sync_copy(src_hbm.at[pl.ds(base + p * PH, PH)], sall)
        pltpu.sync_copy(dst_hbm.at[pl.ds(base + p * PH, PH)], dall)

        def pair_body(i, carry):
            o0 = (2 * i) * CHUNK
            regcopy(sidx0, sall, o0)
            cp0 = pltpu.async_copy(y_hbm.at[sidx0], rows0, gsem0)
            regcopy(sidx1, sall, o0 + CHUNK)
            cp1 = pltpu.async_copy(y_hbm.at[sidx1], rows1, gsem1)
            regcopy(didx0, dall, o0)
            regcopy(didx1, dall, o0 + CHUNK)
            cp0.wait()
            sc0 = pltpu.async_copy(rows0, z_sh.at[didx0], ssem0, add=True)
            cp1.wait()
            sc1 = pltpu.async_copy(rows1, z_sh.at[didx1], ssem1, add=True)
            sc0.wait()
            sc1.wait()
            return carry

        lax.fori_loop(0, PCH // 2, pair_body, 0)

        o = (PCH - 1) * CHUNK
        regcopy(sidx0, sall, o)
        cp = pltpu.async_copy(y_hbm.at[sidx0], rows0, gsem0)
        regcopy(didx0, dall, o)
        cp.wait()
        pltpu.sync_copy(rows0, z_sh.at[didx0], add=True)

    off = base + NFULL * CHUNK
    pltpu.sync_copy(src_hbm.at[pl.ds(off, TAIL)], sidx_t)
    cp = pltpu.async_copy(y_hbm.at[sidx_t], rows_t, gsem0)
    pltpu.sync_copy(dst_hbm.at[pl.ds(off, TAIL)], didx_t)
    cp.wait()
    pltpu.sync_copy(rows_t, z_sh.at[didx_t], add=True)

    plsc.subcore_barrier()
    pltpu.sync_copy(z_sh.at[pl.ds(row0, RPT)], out_hbm.at[c, pl.ds(row0, RPT)])


_edge_call = pl.kernel(
    _edge_body,
    out_type=jax.ShapeDtypeStruct((NC, NPAD, DH), jnp.float32),
    mesh=_mesh,
    scratch_types=[
        pltpu.VMEM_SHARED((NPAD, DH), jnp.float32),
        pltpu.VMEM((PH,), jnp.int32),
        pltpu.VMEM((PH,), jnp.int32),
        pltpu.VMEM((CHUNK,), jnp.int32),
        pltpu.VMEM((CHUNK,), jnp.int32),
        pltpu.VMEM((CHUNK,), jnp.int32),
        pltpu.VMEM((CHUNK,), jnp.int32),
        pltpu.VMEM((CHUNK, DH), jnp.float32),
        pltpu.VMEM((CHUNK, DH), jnp.float32),
        pltpu.VMEM((TAIL,), jnp.int32),
        pltpu.VMEM((TAIL,), jnp.int32),
        pltpu.VMEM((TAIL, DH), jnp.float32),
        pltpu.SemaphoreType.DMA,
        pltpu.SemaphoreType.DMA,
        pltpu.SemaphoreType.DMA,
        pltpu.SemaphoreType.DMA,
    ],
)


def _tca_body(x_ref, w_ref, degt_ref, y_ref):
    deg = jnp.sum(degt_ref[...], axis=1, keepdims=True) + 1.0
    dinv = lax.rsqrt(deg)
    xw = jnp.dot(x_ref[...], w_ref[...], preferred_element_type=jnp.float32)
    y_ref[...] = xw * dinv


def _tca(x, w, degt):
    return pl.pallas_call(
        _tca_body,
        grid=(N // BLK,),
        in_specs=[
            pl.BlockSpec((BLK, DIN), lambda i: (i, 0)),
            pl.BlockSpec((DIN, DH), lambda i: (0, 0)),
            pl.BlockSpec((BLK, NW), lambda i: (i, 0)),
        ],
        out_specs=pl.BlockSpec((BLK, DH), lambda i: (i, 0)),
        out_shape=jax.ShapeDtypeStruct((NPAD, DH), jnp.float32),
    )(x, w, degt)


def _tcb_body(z_ref, y_ref, degt_ref, bconv_ref, batch_ref,
              wlin_ref, blin_ref, out_ref, sums, cnt):
    i = pl.program_id(0)

    @pl.when(i == 0)
    def _():
        sums[...] = jnp.zeros_like(sums)
        cnt[...] = jnp.zeros_like(cnt)

    deg = jnp.sum(degt_ref[...], axis=1, keepdims=True) + 1.0
    dinv = lax.rsqrt(deg)
    z = z_ref[0] + z_ref[1]
    h = dinv * (z + y_ref[...]) + bconv_ref[...]
    h = jnp.maximum(h, 0.0)
    gid = lax.broadcasted_iota(jnp.int32, (BLK, G), 1)
    onehot = (batch_ref[...] == gid).astype(jnp.float32)
    sums[...] += lax.dot_general(onehot, h, (((0,), (0,)), ((), ())),
                                 preferred_element_type=jnp.float32)
    cnt[...] += lax.dot_general(onehot, jnp.ones((BLK, 1), jnp.float32),
                                (((0,), (0,)), ((), ())),
                                preferred_element_type=jnp.float32)

    @pl.when(i == pl.num_programs(0) - 1)
    def _():
        emb = sums[...] / jnp.maximum(cnt[...], 1.0)
        out_ref[...] = jnp.tanh(
            jnp.dot(emb, wlin_ref[...], preferred_element_type=jnp.float32)
            + blin_ref[...])


def _tcb(zp, y, degt, bconv, batch2, wlin, blin):
    return pl.pallas_call(
        _tcb_body,
        grid=(N // BLK,),
        in_specs=[
            pl.BlockSpec((NC, BLK, DH), lambda i: (0, i, 0)),
            pl.BlockSpec((BLK, DH), lambda i: (i, 0)),
            pl.BlockSpec((BLK, NW), lambda i: (i, 0)),
            pl.BlockSpec((1, DH), lambda i: (0, 0)),
            pl.BlockSpec((BLK, 1), lambda i: (i, 0)),
            pl.BlockSpec((DH, DOUT), lambda i: (0, 0)),
            pl.BlockSpec((1, DOUT), lambda i: (0, 0)),
        ],
        out_specs=pl.BlockSpec((G, DOUT), lambda i: (0, 0)),
        out_shape=jax.ShapeDtypeStruct((G, DOUT), jnp.float32),
        scratch_shapes=[
            pltpu.VMEM((G, DH), jnp.float32),
            pltpu.VMEM((G, 1), jnp.float32),
        ],
    )(zp, y, degt, bconv, batch2, wlin, blin)


@jax.jit
def kernel(x, edge_index, batch, W_conv, b_conv, W_lin, b_lin):
    src = edge_index[0].astype(jnp.int32)
    dst = edge_index[1].astype(jnp.int32)

    degp = _deg_call(dst)
    degt = degp.T

    y = _tca(x, W_conv, degt)

    zp = _edge_call(src, dst, y)

    batch2 = batch.astype(jnp.int32).reshape(N, 1)
    bconv = b_conv.reshape(1, DH)
    blin = b_lin.reshape(1, DOUT)
    return _tcb(zp, y, degt, bconv, batch2, W_lin, blin)


# speedup vs baseline: 1.3221x; 1.2029x over previous
import jax
import jax.numpy as jnp
from jax import lax
from jax.experimental import pallas as pl
from jax.experimental.pallas import tpu as pltpu
from jax.experimental.pallas import tpu_sc as plsc

N = 10000
E = 320000
DIN = 128
DH = 128
DOUT = 64
G = 64

NC = 2
NS = 16
NW = NC * NS
EC = E // NW
CHUNK = 64
NFULL = EC // CHUNK
TAIL = EC - NFULL * CHUNK
PCH = NFULL // 2
PH = PCH * CHUNK
PGRP = (PCH - 2) // 4
NPAD = 10240
RPT = NPAD // NS
BLK = 2000

_mesh = plsc.VectorSubcoreMesh(core_axis_name="c", subcore_axis_name="s")


def _deg_body(dst_hbm, out_hbm, dst_v, deg_v):
    c = lax.axis_index("c")
    s = lax.axis_index("s")
    wid = s * NC + c
    base = wid * EC
    pltpu.sync_copy(dst_hbm.at[pl.ds(base, EC)], dst_v)
    z16 = jnp.zeros((16,), jnp.float32)
    ones16 = jnp.ones((16,), jnp.float32)

    def zero_body(i, carry):
        deg_v[pl.ds(i * 16, 16)] = z16
        return carry

    lax.fori_loop(0, NPAD // 16, zero_body, 0)

    def acc_body(i, carry):
        idx = dst_v[pl.ds(i * 16, 16)]
        plsc.addupdate_scatter(deg_v, [idx], ones16)
        return carry

    lax.fori_loop(0, EC // 16, acc_body, 0)
    pltpu.sync_copy(deg_v, out_hbm.at[wid])


_deg_call = pl.kernel(
    _deg_body,
    out_type=jax.ShapeDtypeStruct((NW, NPAD), jnp.float32),
    mesh=_mesh,
    compiler_params=pltpu.CompilerParams(needs_layout_passes=False),
    scratch_types=[
        pltpu.VMEM((EC,), jnp.int32),
        pltpu.VMEM((NPAD,), jnp.float32),
    ],
)


def _edge_body(src_hbm, dst_hbm, y_hbm, out_hbm, z_sh, sall, dall,
               sa0, da0, sa1, da1, sb0, db0, sb1, db1,
               ra0, ra1, rb0, rb1, sidx_t, didx_t, rows_t,
               gsa0, gsa1, gsb0, gsb1, ssa0, ssa1, ssb0, ssb1):
    c = lax.axis_index("c")
    s = lax.axis_index("s")
    wid = s * NC + c
    base = wid * EC

    z16 = jnp.zeros((16,), jnp.float32)

    def zrow(j, carry):
        def zcol(k, c2):
            ra0[j, pl.ds(k * 16, 16)] = z16
            return c2

        lax.fori_loop(0, DH // 16, zcol, 0)
        return carry

    lax.fori_loop(0, CHUNK, zrow, 0)
    row0 = s * RPT
    for r in range(RPT // CHUNK):
        pltpu.sync_copy(ra0, z_sh.at[pl.ds(row0 + r * CHUNK, CHUNK)])

    plsc.subcore_barrier()

    def regcopy(dst_small, src_big, off):
        for k in range(CHUNK // 16):
            dst_small[pl.ds(k * 16, 16)] = src_big[pl.ds(off + k * 16, 16)]

    def fetch_a(ci):
        regcopy(sa0, sall, ci * CHUNK)
        regcopy(sa1, sall, (ci + 1) * CHUNK)
        regcopy(da0, dall, ci * CHUNK)
        regcopy(da1, dall, (ci + 1) * CHUNK)
        cp0 = pltpu.async_copy(y_hbm.at[sa0], ra0, gsa0)
        cp1 = pltpu.async_copy(y_hbm.at[sa1], ra1, gsa1)
        return cp0, cp1

    for p in range(2):
        pltpu.sync_copy(src_hbm.at[pl.ds(base + p * PH, PH)], sall)
        pltpu.sync_copy(dst_hbm.at[pl.ds(base + p * PH, PH)], dall)

        fetch_a(0)

        def group_body(i, carry):
            c0 = 4 * i
            regcopy(sb0, sall, (c0 + 2) * CHUNK)
            regcopy(sb1, sall, (c0 + 3) * CHUNK)
            regcopy(db0, dall, (c0 + 2) * CHUNK)
            regcopy(db1, dall, (c0 + 3) * CHUNK)
            pltpu.make_async_copy(y_hbm.at[sa0], ra0, gsa0).wait()
            w0 = pltpu.async_copy(ra0, z_sh.at[da0], ssa0, add=True)
            pltpu.make_async_copy(y_hbm.at[sa1], ra1, gsa1).wait()
            w1 = pltpu.async_copy(ra1, z_sh.at[da1], ssa1, add=True)
            cpb0 = pltpu.async_copy(y_hbm.at[sb0], rb0, gsb0)
            cpb1 = pltpu.async_copy(y_hbm.at[sb1], rb1, gsb1)
            w0.wait()
            w1.wait()
            fetch_a(c0 + 4)
            cpb0.wait()
            wb0 = pltpu.async_copy(rb0, z_sh.at[db0], ssb0, add=True)
            cpb1.wait()
            wb1 = pltpu.async_copy(rb1, z_sh.at[db1], ssb1, add=True)
            wb0.wait()
            wb1.wait()
            return carry

        lax.fori_loop(0, PGRP, group_body, 0)

        pltpu.make_async_copy(y_hbm.at[sa0], ra0, gsa0).wait()
        pltpu.sync_copy(ra0, z_sh.at[da0], add=True)
        pltpu.make_async_copy(y_hbm.at[sa1], ra1, gsa1).wait()
        pltpu.sync_copy(ra1, z_sh.at[da1], add=True)

    off = base + NFULL * CHUNK
    pltpu.sync_copy(src_hbm.at[pl.ds(off, TAIL)], sidx_t)
    cp = pltpu.async_copy(y_hbm.at[sidx_t], rows_t, gsa0)
    pltpu.sync_copy(dst_hbm.at[pl.ds(off, TAIL)], didx_t)
    cp.wait()
    pltpu.sync_copy(rows_t, z_sh.at[didx_t], add=True)

    plsc.subcore_barrier()
    pltpu.sync_copy(z_sh.at[pl.ds(row0, RPT)], out_hbm.at[c, pl.ds(row0, RPT)])


_edge_call = pl.kernel(
    _edge_body,
    out_type=jax.ShapeDtypeStruct((NC, NPAD, DH), jnp.float32),
    mesh=_mesh,
    scratch_types=(
        [pltpu.VMEM_SHARED((NPAD, DH), jnp.float32)]
        + [pltpu.VMEM((PH,), jnp.int32)] * 2
        + [pltpu.VMEM((CHUNK,), jnp.int32)] * 8
        + [pltpu.VMEM((CHUNK, DH), jnp.float32)] * 4
        + [pltpu.VMEM((TAIL,), jnp.int32)] * 2
        + [pltpu.VMEM((TAIL, DH), jnp.float32)]
        + [pltpu.SemaphoreType.DMA] * 8
    ),
)


def _tca_body(x_ref, w_ref, degt_ref, y_ref):
    deg = jnp.sum(degt_ref[...], axis=1, keepdims=True) + 1.0
    dinv = lax.rsqrt(deg)
    xw = jnp.dot(x_ref[...], w_ref[...], preferred_element_type=jnp.float32)
    y_ref[...] = xw * dinv


def _tca(x, w, degt):
    return pl.pallas_call(
        _tca_body,
        grid=(N // BLK,),
        in_specs=[
            pl.BlockSpec((BLK, DIN), lambda i: (i, 0)),
            pl.BlockSpec((DIN, DH), lambda i: (0, 0)),
            pl.BlockSpec((BLK, NW), lambda i: (i, 0)),
        ],
        out_specs=pl.BlockSpec((BLK, DH), lambda i: (i, 0)),
        out_shape=jax.ShapeDtypeStruct((NPAD, DH), jnp.float32),
    )(x, w, degt)


def _tcb_body(z_ref, y_ref, degt_ref, bconv_ref, batch_ref,
              wlin_ref, blin_ref, out_ref, sums, cnt):
    i = pl.program_id(0)

    @pl.when(i == 0)
    def _():
        sums[...] = jnp.zeros_like(sums)
        cnt[...] = jnp.zeros_like(cnt)

    deg = jnp.sum(degt_ref[...], axis=1, keepdims=True) + 1.0
    dinv = lax.rsqrt(deg)
    z = z_ref[0] + z_ref[1]
    h = dinv * (z + y_ref[...]) + bconv_ref[...]
    h = jnp.maximum(h, 0.0)
    gid = lax.broadcasted_iota(jnp.int32, (BLK, G), 1)
    onehot = (batch_ref[...] == gid).astype(jnp.float32)
    sums[...] += lax.dot_general(onehot, h, (((0,), (0,)), ((), ())),
                                 preferred_element_type=jnp.float32)
    cnt[...] += lax.dot_general(onehot, jnp.ones((BLK, 1), jnp.float32),
                                (((0,), (0,)), ((), ())),
                                preferred_element_type=jnp.float32)

    @pl.when(i == pl.num_programs(0) - 1)
    def _():
        emb = sums[...] / jnp.maximum(cnt[...], 1.0)
        out_ref[...] = jnp.tanh(
            jnp.dot(emb, wlin_ref[...], preferred_element_type=jnp.float32)
            + blin_ref[...])


def _tcb(zp, y, degt, bconv, batch2, wlin, blin):
    return pl.pallas_call(
        _tcb_body,
        grid=(N // BLK,),
        in_specs=[
            pl.BlockSpec((NC, BLK, DH), lambda i: (0, i, 0)),
            pl.BlockSpec((BLK, DH), lambda i: (i, 0)),
            pl.BlockSpec((BLK, NW), lambda i: (i, 0)),
            pl.BlockSpec((1, DH), lambda i: (0, 0)),
            pl.BlockSpec((BLK, 1), lambda i: (i, 0)),
            pl.BlockSpec((DH, DOUT), lambda i: (0, 0)),
            pl.BlockSpec((1, DOUT), lambda i: (0, 0)),
        ],
        out_specs=pl.BlockSpec((G, DOUT), lambda i: (0, 0)),
        out_shape=jax.ShapeDtypeStruct((G, DOUT), jnp.float32),
        scratch_shapes=[
            pltpu.VMEM((G, DH), jnp.float32),
            pltpu.VMEM((G, 1), jnp.float32),
        ],
    )(zp, y, degt, bconv, batch2, wlin, blin)


@jax.jit
def kernel(x, edge_index, batch, W_conv, b_conv, W_lin, b_lin):
    src = edge_index[0].astype(jnp.int32)
    dst = edge_index[1].astype(jnp.int32)

    degp = _deg_call(dst)
    degt = degp.T

    y = _tca(x, W_conv, degt)

    zp = _edge_call(src, dst, y)

    batch2 = batch.astype(jnp.int32).reshape(N, 1)
    bconv = b_conv.reshape(1, DH)
    blin = b_lin.reshape(1, DOUT)
    return _tcb(zp, y, degt, bconv, batch2, W_lin, blin)
